# baseline (speedup 1.0000x reference)
import functools

import jax
import jax.numpy as jnp
from jax.experimental import pallas as pl
from jax.experimental.pallas import tpu as pltpu


def _fold_groups(x, lane_groups, rem, m, s):
    for k in range(lane_groups):
        part = x[:, :, 128 * k:128 * (k + 1)]
        m = part if m is None else jnp.maximum(m, part)
        s = part if s is None else s + part
    if rem:
        tail = x[:, :, 128 * lane_groups:]
        tm = jnp.max(tail, axis=-1, keepdims=True)
        ts = jnp.sum(tail, axis=-1, keepdims=True)
        m = tm if m is None else jnp.maximum(m, tm)
        if s is None:
            s = ts
        else:
            s = jnp.concatenate([s[:, :, 0:1] + ts, s[:, :, 1:]], axis=-1)
    return m, s


def _pool_body_lsplit(*refs, c, lane_groups, rem, inv_l):
    o_ref = refs[-1]
    m = None
    s = None
    for x_ref in refs[:-1]:
        x = x_ref[...].astype(jnp.float32)
        m, s = _fold_groups(x, lane_groups, rem, m, s)
    o_ref[:, :c] = jnp.max(m, axis=-1).astype(o_ref.dtype)
    o_ref[:, c:] = (jnp.sum(s, axis=-1) * inv_l).astype(o_ref.dtype)


def kernel(x):
    N, C, L = x.shape
    cost = pl.CostEstimate(
        flops=2 * N * C * L,
        transcendentals=0,
        bytes_accessed=N * C * L * x.dtype.itemsize + N * 2 * C * x.dtype.itemsize,
    )

    tb = 8
    while tb > 1 and N % tb != 0:
        tb -= 1
    streams = 2 if (L % 256 == 0 and N // tb >= 2) else 1
    tl = L // streams
    lane_groups = tl // 128
    rem = tl % 128

    in_specs = [
        pl.BlockSpec((tb, C, tl),
                     functools.partial(lambda j, i: (i, 0, j), j))
        for j in range(streams)
    ]
    out = pl.pallas_call(
        functools.partial(_pool_body_lsplit, c=C, lane_groups=lane_groups,
                          rem=rem, inv_l=float(1.0 / L)),
        out_shape=jax.ShapeDtypeStruct((N, 2 * C), x.dtype),
        grid=(N // tb,),
        in_specs=in_specs,
        out_specs=pl.BlockSpec((tb, 2 * C), lambda i: (i, 0)),
        compiler_params=pltpu.CompilerParams(
            dimension_semantics=("parallel",),
            vmem_limit_bytes=48 << 20,
        ),
        cost_estimate=cost,
    )(*([x] * streams))
    return out.reshape(N, 2 * C, 1)


# speedup vs baseline: 1.2147x; 1.2147x over previous
import functools

import jax
import jax.numpy as jnp
from jax.experimental import pallas as pl
from jax.experimental.pallas import tpu as pltpu


def _pool_one(x, o_ref, rows, *, c, lane_groups, rem, inv_l):
    if lane_groups >= 1:
        m = x[:, :, 0:128]
        s = m
        for k in range(1, lane_groups):
            part = x[:, :, 128 * k:128 * (k + 1)]
            m = jnp.maximum(m, part)
            s = s + part
        mx = jnp.max(m, axis=-1)
        sm = jnp.sum(s, axis=-1)
        if rem:
            tail = x[:, :, 128 * lane_groups:]
            mx = jnp.maximum(mx, jnp.max(tail, axis=-1))
            sm = sm + jnp.sum(tail, axis=-1)
    else:
        mx = jnp.max(x, axis=-1)
        sm = jnp.sum(x, axis=-1)
    o_ref[rows, :c] = mx.astype(o_ref.dtype)
    o_ref[rows, c:] = (sm * inv_l).astype(o_ref.dtype)


def _pool_body(*refs, c, tb, lane_groups, rem, inv_l):
    x_refs = refs[:-1]
    o_ref = refs[-1]
    for j, x_ref in enumerate(x_refs):
        x = x_ref[...].astype(jnp.float32)
        _pool_one(x, o_ref, pl.ds(j * tb, tb), c=c, lane_groups=lane_groups,
                  rem=rem, inv_l=inv_l)


def kernel(x):
    N, C, L = x.shape
    lane_groups = L // 128
    rem = L % 128
    cost = pl.CostEstimate(
        flops=2 * N * C * L,
        transcendentals=0,
        bytes_accessed=N * C * L * x.dtype.itemsize + N * 2 * C * x.dtype.itemsize,
    )

    tb, streams = 8, 2
    if N % (streams * tb) != 0 or N // (streams * tb) < 2:
        streams = 1
        tb = max(1, min(N, 16))
        while N % tb != 0:
            tb -= 1

    in_specs = [
        pl.BlockSpec((tb, C, L),
                     functools.partial(lambda j, i: (streams * i + j, 0, 0), j))
        for j in range(streams)
    ]
    out = pl.pallas_call(
        functools.partial(_pool_body, c=C, tb=tb, lane_groups=lane_groups,
                          rem=rem, inv_l=float(1.0 / L)),
        out_shape=jax.ShapeDtypeStruct((N, 2 * C), x.dtype),
        grid=(N // (streams * tb),),
        in_specs=in_specs,
        out_specs=pl.BlockSpec((streams * tb, 2 * C), lambda i: (i, 0)),
        compiler_params=pltpu.CompilerParams(
            dimension_semantics=("parallel",),
            vmem_limit_bytes=48 << 20,
        ),
        cost_estimate=cost,
    )(*([x] * streams))
    return out.reshape(N, 2 * C, 1)
